# baseline (speedup 1.0000x reference)
import functools

import jax
import jax.numpy as jnp
from jax import lax
from jax.experimental import pallas as pl
from jax.experimental.pallas import tpu as pltpu
from jax.experimental.pallas import tpu_sc as plsc

H = 128
H2 = 256
H3 = 384

NC = 2
NS = 16
NW = NC * NS

CH = 128

_DN = (((1,), (1,)), ((), ()))


def _node_pre_body(x_ref, w1_ref, b1_ref, s1_ref, s2_ref):
    xb = x_ref[...]
    w1 = w1_ref[...]
    s1_ref[...] = lax.dot_general(xb, w1[:, 0:H], _DN,
                                  preferred_element_type=jnp.float32) + b1_ref[...]
    s2_ref[...] = lax.dot_general(xb, w1[:, H:H2], _DN,
                                  preferred_element_type=jnp.float32)


def _edge_mlp_body(g1_ref, g2_ref, ea_ref, w1c_ref, w2_ref, b2_ref, out_ref):
    t = g1_ref[...] + g2_ref[...] + lax.dot_general(
        ea_ref[...], w1c_ref[...], _DN, preferred_element_type=jnp.float32)
    h = jnp.maximum(t, 0.0)
    out_ref[...] = lax.dot_general(
        h, w2_ref[...], _DN, preferred_element_type=jnp.float32) + b2_ref[...]


def _make_gru_body(n_agg):
    def _gru_body(*refs):
        agg_refs = refs[:n_agg]
        x_ref, wih_ref, whh_ref, bih_ref, bhh_ref, out_ref = refs[n_agg:]
        agg = functools.reduce(lambda a, r: a + r[...], agg_refs[1:],
                               agg_refs[0][...])
        xb = x_ref[...]
        gi = lax.dot_general(agg, wih_ref[...], _DN,
                             preferred_element_type=jnp.float32) + bih_ref[...]
        gh = lax.dot_general(xb, whh_ref[...], _DN,
                             preferred_element_type=jnp.float32) + bhh_ref[...]
        r = jax.nn.sigmoid(gi[:, 0:H] + gh[:, 0:H])
        z = jax.nn.sigmoid(gi[:, H:H2] + gh[:, H:H2])
        n = jnp.tanh(gi[:, H2:H3] + r * gh[:, H2:H3])
        out_ref[...] = (1.0 - z) * n + z * xb

    return _gru_body


def _chunk_deal(wid, nchunks):
    base_ct = nchunks // NW
    rem = nchunks % NW
    extra = (wid < rem).astype(jnp.int32)
    start = base_ct * wid + jnp.minimum(wid, rem)
    return start, base_ct + extra


def _make_gather(n_nodes, n_edges, chunk_off=0):
    nchunks = n_edges // CH
    assert n_edges % CH == 0
    mesh = plsc.VectorSubcoreMesh(core_axis_name="c", subcore_axis_name="s")

    @functools.partial(
        pl.kernel, mesh=mesh,
        out_type=[jax.ShapeDtypeStruct((n_edges, H), jnp.float32),
                  jax.ShapeDtypeStruct((n_edges, H), jnp.float32)],
        scratch_types=[pltpu.VMEM((CH,), jnp.int32),
                       pltpu.VMEM((CH,), jnp.int32),
                       pltpu.VMEM((CH,), jnp.int32),
                       pltpu.VMEM((CH,), jnp.int32),
                       pltpu.VMEM((CH, H), jnp.float32),
                       pltpu.VMEM((CH, H), jnp.float32),
                       pltpu.VMEM((CH, H), jnp.float32),
                       pltpu.VMEM((CH, H), jnp.float32),
                       pltpu.SemaphoreType.DMA,
                       pltpu.SemaphoreType.DMA],
    )
    def gather_k(s1_hbm, s2_hbm, src_hbm, dst_hbm, o1_hbm, o2_hbm,
                 ia1, ia2, ib1, ib2, ba1, ba2, bb1, bb2, gsem, wsem):
        wid = lax.axis_index("s") * NC + lax.axis_index("c")
        start, my_ct = _chunk_deal(wid, nchunks)

        def do_chunk(base, idx1, idx2, buf1, buf2):
            gbase = base + chunk_off * CH
            pltpu.sync_copy(src_hbm.at[pl.ds(gbase, CH)], idx1)
            pltpu.sync_copy(dst_hbm.at[pl.ds(gbase, CH)], idx2)
            g1 = pltpu.async_copy(s1_hbm.at[idx1], buf1, gsem)
            g2 = pltpu.async_copy(s2_hbm.at[idx2], buf2, gsem)
            return g1, g2

        def put_chunk(base, buf1, buf2):
            w1 = pltpu.async_copy(buf1, o1_hbm.at[pl.ds(base, CH)], wsem)
            w2 = pltpu.async_copy(buf2, o2_hbm.at[pl.ds(base, CH)], wsem)
            return w1, w2

        def pair(p, carry):
            c0 = (start + 2 * p) * CH
            c1 = c0 + CH
            ga1, ga2 = do_chunk(c0, ia1, ia2, ba1, ba2)
            gb1, gb2 = do_chunk(c1, ib1, ib2, bb1, bb2)
            ga1.wait()
            ga2.wait()
            wa1, wa2 = put_chunk(c0, ba1, ba2)
            gb1.wait()
            gb2.wait()
            wb1, wb2 = put_chunk(c1, bb1, bb2)
            wa1.wait()
            wa2.wait()
            wb1.wait()
            wb2.wait()
            return carry

        lax.fori_loop(0, my_ct // 2, pair, 0)

        @pl.when(my_ct % 2 == 1)
        def _():
            c0 = (start + my_ct - 1) * CH
            g1, g2 = do_chunk(c0, ia1, ia2, ba1, ba2)
            g1.wait()
            g2.wait()
            w1, w2 = put_chunk(c0, ba1, ba2)
            w1.wait()
            w2.wait()

    return gather_k


def _make_scatter(n_pad, n_edges, chunk_off=0):
    nchunks = n_edges // CH
    rpt = n_pad // NS
    assert n_edges % CH == 0 and n_pad % NS == 0 and rpt % 8 == 0
    mesh = plsc.VectorSubcoreMesh(core_axis_name="c", subcore_axis_name="s")

    @functools.partial(
        pl.kernel, mesh=mesh,
        out_type=[jax.ShapeDtypeStruct((n_pad, H), jnp.float32),
                  jax.ShapeDtypeStruct((n_pad, H), jnp.float32)],
        scratch_types=[pltpu.VMEM((CH,), jnp.int32),
                       pltpu.VMEM((CH,), jnp.int32),
                       pltpu.VMEM((CH, H), jnp.float32),
                       pltpu.VMEM((CH, H), jnp.float32),
                       pltpu.VMEM_SHARED((n_pad, H), jnp.float32),
                       pltpu.SemaphoreType.DMA],
    )
    def scatter_k(msg_hbm, dst_hbm, zeros_hbm, agg0_hbm, agg1_hbm,
                  ia, ib, ma, mb, acc_spmem, ssem):
        cid = lax.axis_index("c")
        sid = lax.axis_index("s")
        wid = sid * NC + cid
        start, my_ct = _chunk_deal(wid, nchunks)

        pltpu.sync_copy(zeros_hbm, acc_spmem.at[pl.ds(sid * rpt, rpt)])
        plsc.subcore_barrier()

        def add_chunk(base, idx, mbuf):
            pltpu.sync_copy(dst_hbm.at[pl.ds(base + chunk_off * CH, CH)], idx)
            pltpu.sync_copy(msg_hbm.at[pl.ds(base, CH)], mbuf)
            return pltpu.async_copy(mbuf, acc_spmem.at[idx], ssem, add=True)

        def pair(p, carry):
            c0 = (start + 2 * p) * CH
            aa = add_chunk(c0, ia, ma)
            ab = add_chunk(c0 + CH, ib, mb)
            aa.wait()
            ab.wait()
            return carry

        lax.fori_loop(0, my_ct // 2, pair, 0)

        @pl.when(my_ct % 2 == 1)
        def _():
            c0 = (start + my_ct - 1) * CH
            add_chunk(c0, ia, ma).wait()

        plsc.subcore_barrier()

        @pl.when(cid == 0)
        def _():
            pltpu.sync_copy(acc_spmem.at[pl.ds(sid * rpt, rpt)],
                            agg0_hbm.at[pl.ds(sid * rpt, rpt)])

        @pl.when(cid == 1)
        def _():
            pltpu.sync_copy(acc_spmem.at[pl.ds(sid * rpt, rpt)],
                            agg1_hbm.at[pl.ds(sid * rpt, rpt)])

    return scatter_k


def kernel(x, edge_index, edge_attr, W1, b1, W2, b2, Wih, Whh, bih, bhh):
    n_nodes, h = x.shape
    n_edges = edge_attr.shape[0]
    assert h == H

    src = edge_index[0].astype(jnp.int32)
    dst = edge_index[1].astype(jnp.int32)

    bn = 2000
    be = 2000
    assert n_nodes % bn == 0 and n_edges % be == 0

    f32 = jnp.float32

    s1t, s2t = pl.pallas_call(
        _node_pre_body,
        grid=(n_nodes // bn,),
        in_specs=[pl.BlockSpec((bn, H), lambda i: (i, 0)),
                  pl.BlockSpec((H, H3), lambda i: (0, 0)),
                  pl.BlockSpec((1, H), lambda i: (0, 0))],
        out_specs=[pl.BlockSpec((bn, H), lambda i: (i, 0)),
                   pl.BlockSpec((bn, H), lambda i: (i, 0))],
        out_shape=[jax.ShapeDtypeStruct((n_nodes, H), f32),
                   jax.ShapeDtypeStruct((n_nodes, H), f32)],
    )(x, W1, b1.reshape(1, H))

    nshard = 4
    n_pad = ((n_nodes + NS * 8 - 1) // (NS * 8)) * (NS * 8)
    zeros = jnp.zeros((n_pad // NS, H), f32)
    n_sh = n_edges // nshard
    assert n_sh % CH == 0 and n_sh % be == 0
    sh_blocks = n_sh // be

    aggs = []
    for s in range(nshard):
        off = s * (n_sh // CH)
        g1, g2 = _make_gather(n_nodes, n_sh, off)(s1t, s2t, src, dst)
        msg = pl.pallas_call(
            _edge_mlp_body,
            grid=(sh_blocks,),
            in_specs=[pl.BlockSpec((be, H), lambda i: (i, 0)),
                      pl.BlockSpec((be, H), lambda i: (i, 0)),
                      pl.BlockSpec((be, H),
                                   lambda i, o=s * sh_blocks: (i + o, 0)),
                      pl.BlockSpec((H, H), lambda i: (0, 0)),
                      pl.BlockSpec((H, H), lambda i: (0, 0)),
                      pl.BlockSpec((1, H), lambda i: (0, 0))],
            out_specs=pl.BlockSpec((be, H), lambda i: (i, 0)),
            out_shape=jax.ShapeDtypeStruct((n_sh, H), f32),
        )(g1, g2, edge_attr, W1[:, H2:H3], W2, b2.reshape(1, H))
        a0, a1 = _make_scatter(n_pad, n_sh, off)(msg, dst, zeros)
        aggs += [a0, a1]

    out = pl.pallas_call(
        _make_gru_body(len(aggs)),
        grid=(n_nodes // bn,),
        in_specs=([pl.BlockSpec((bn, H), lambda i: (i, 0))] * (len(aggs) + 1)
                  + [pl.BlockSpec((H3, H), lambda i: (0, 0)),
                     pl.BlockSpec((H3, H), lambda i: (0, 0)),
                     pl.BlockSpec((1, H3), lambda i: (0, 0)),
                     pl.BlockSpec((1, H3), lambda i: (0, 0))]),
        out_specs=pl.BlockSpec((bn, H), lambda i: (i, 0)),
        out_shape=jax.ShapeDtypeStruct((n_nodes, H), f32),
    )(*aggs, x, Wih, Whh, bih.reshape(1, H3), bhh.reshape(1, H3))

    return out


# speedup vs baseline: 1.1129x; 1.1129x over previous
import functools

import jax
import jax.numpy as jnp
from jax import lax
from jax.experimental import pallas as pl
from jax.experimental.pallas import tpu as pltpu
from jax.experimental.pallas import tpu_sc as plsc

H = 128
H2 = 256
H3 = 384

NC = 2
NS = 16
NW = NC * NS

CH = 128

_DN = (((1,), (1,)), ((), ()))


def _node_pre_body(x_ref, w1_ref, b1_ref, s1_ref, s2_ref):
    xb = x_ref[...]
    w1 = w1_ref[...]
    s1_ref[...] = lax.dot_general(xb, w1[:, 0:H], _DN,
                                  preferred_element_type=jnp.float32) + b1_ref[...]
    s2_ref[...] = lax.dot_general(xb, w1[:, H:H2], _DN,
                                  preferred_element_type=jnp.float32)


def _edge_mlp_body(g1_ref, g2_ref, ea_ref, w1c_ref, w2_ref, b2_ref, out_ref):
    t = g1_ref[...] + g2_ref[...] + lax.dot_general(
        ea_ref[...], w1c_ref[...], _DN, preferred_element_type=jnp.float32)
    h = jnp.maximum(t, 0.0)
    out_ref[...] = lax.dot_general(
        h, w2_ref[...], _DN, preferred_element_type=jnp.float32) + b2_ref[...]


def _make_gru_body(n_agg):
    def _gru_body(*refs):
        agg_refs = refs[:n_agg]
        x_ref, wih_ref, whh_ref, bih_ref, bhh_ref, out_ref = refs[n_agg:]
        agg = functools.reduce(lambda a, r: a + r[...], agg_refs[1:],
                               agg_refs[0][...])
        xb = x_ref[...]
        gi = lax.dot_general(agg, wih_ref[...], _DN,
                             preferred_element_type=jnp.float32) + bih_ref[...]
        gh = lax.dot_general(xb, whh_ref[...], _DN,
                             preferred_element_type=jnp.float32) + bhh_ref[...]
        r = jax.nn.sigmoid(gi[:, 0:H] + gh[:, 0:H])
        z = jax.nn.sigmoid(gi[:, H:H2] + gh[:, H:H2])
        n = jnp.tanh(gi[:, H2:H3] + r * gh[:, H2:H3])
        out_ref[...] = (1.0 - z) * n + z * xb

    return _gru_body


def _chunk_deal(wid, nchunks):
    base_ct = nchunks // NW
    rem = nchunks % NW
    extra = (wid < rem).astype(jnp.int32)
    start = base_ct * wid + jnp.minimum(wid, rem)
    return start, base_ct + extra


def _make_gather(n_nodes, n_edges, chunk_off=0):
    nchunks = n_edges // CH
    assert n_edges % CH == 0
    mesh = plsc.VectorSubcoreMesh(core_axis_name="c", subcore_axis_name="s")

    @functools.partial(
        pl.kernel, mesh=mesh,
        out_type=[jax.ShapeDtypeStruct((n_edges, H), jnp.float32),
                  jax.ShapeDtypeStruct((n_edges, H), jnp.float32)],
        scratch_types=[pltpu.VMEM((CH,), jnp.int32),
                       pltpu.VMEM((CH,), jnp.int32),
                       pltpu.VMEM((CH,), jnp.int32),
                       pltpu.VMEM((CH,), jnp.int32),
                       pltpu.VMEM((CH, H), jnp.float32),
                       pltpu.VMEM((CH, H), jnp.float32),
                       pltpu.VMEM((CH, H), jnp.float32),
                       pltpu.VMEM((CH, H), jnp.float32),
                       pltpu.SemaphoreType.DMA,
                       pltpu.SemaphoreType.DMA],
    )
    def gather_k(s1_hbm, s2_hbm, src_hbm, dst_hbm, o1_hbm, o2_hbm,
                 ia1, ia2, ib1, ib2, ba1, ba2, bb1, bb2, gsem, wsem):
        wid = lax.axis_index("s") * NC + lax.axis_index("c")
        start, my_ct = _chunk_deal(wid, nchunks)

        def do_chunk(base, idx1, idx2, buf1, buf2):
            gbase = base + chunk_off * CH
            pltpu.sync_copy(src_hbm.at[pl.ds(gbase, CH)], idx1)
            pltpu.sync_copy(dst_hbm.at[pl.ds(gbase, CH)], idx2)
            g1 = pltpu.async_copy(s1_hbm.at[idx1], buf1, gsem)
            g2 = pltpu.async_copy(s2_hbm.at[idx2], buf2, gsem)
            return g1, g2

        def put_chunk(base, buf1, buf2):
            w1 = pltpu.async_copy(buf1, o1_hbm.at[pl.ds(base, CH)], wsem)
            w2 = pltpu.async_copy(buf2, o2_hbm.at[pl.ds(base, CH)], wsem)
            return w1, w2

        def pair(p, carry):
            c0 = (start + 2 * p) * CH
            c1 = c0 + CH
            ga1, ga2 = do_chunk(c0, ia1, ia2, ba1, ba2)
            gb1, gb2 = do_chunk(c1, ib1, ib2, bb1, bb2)
            ga1.wait()
            ga2.wait()
            wa1, wa2 = put_chunk(c0, ba1, ba2)
            gb1.wait()
            gb2.wait()
            wb1, wb2 = put_chunk(c1, bb1, bb2)
            wa1.wait()
            wa2.wait()
            wb1.wait()
            wb2.wait()
            return carry

        lax.fori_loop(0, my_ct // 2, pair, 0)

        @pl.when(my_ct % 2 == 1)
        def _():
            c0 = (start + my_ct - 1) * CH
            g1, g2 = do_chunk(c0, ia1, ia2, ba1, ba2)
            g1.wait()
            g2.wait()
            w1, w2 = put_chunk(c0, ba1, ba2)
            w1.wait()
            w2.wait()

    return gather_k


def _make_scatter(n_pad, n_edges, chunk_off=0):
    nchunks = n_edges // CH
    rpt = n_pad // NS
    assert n_edges % CH == 0 and n_pad % NS == 0 and rpt % 8 == 0
    mesh = plsc.VectorSubcoreMesh(core_axis_name="c", subcore_axis_name="s")

    @functools.partial(
        pl.kernel, mesh=mesh,
        out_type=[jax.ShapeDtypeStruct((n_pad, H), jnp.float32),
                  jax.ShapeDtypeStruct((n_pad, H), jnp.float32)],
        scratch_types=[pltpu.VMEM((CH,), jnp.int32),
                       pltpu.VMEM((CH,), jnp.int32),
                       pltpu.VMEM((CH, H), jnp.float32),
                       pltpu.VMEM((CH, H), jnp.float32),
                       pltpu.VMEM_SHARED((n_pad, H), jnp.float32),
                       pltpu.SemaphoreType.DMA],
    )
    def scatter_k(msg_hbm, dst_hbm, zeros_hbm, agg0_hbm, agg1_hbm,
                  ia, ib, ma, mb, acc_spmem, ssem):
        cid = lax.axis_index("c")
        sid = lax.axis_index("s")
        wid = sid * NC + cid
        start, my_ct = _chunk_deal(wid, nchunks)

        pltpu.sync_copy(zeros_hbm, acc_spmem.at[pl.ds(sid * rpt, rpt)])
        plsc.subcore_barrier()

        def add_chunk(base, idx, mbuf):
            pltpu.sync_copy(dst_hbm.at[pl.ds(base + chunk_off * CH, CH)], idx)
            pltpu.sync_copy(msg_hbm.at[pl.ds(base, CH)], mbuf)
            return pltpu.async_copy(mbuf, acc_spmem.at[idx], ssem, add=True)

        def pair(p, carry):
            c0 = (start + 2 * p) * CH
            aa = add_chunk(c0, ia, ma)
            ab = add_chunk(c0 + CH, ib, mb)
            aa.wait()
            ab.wait()
            return carry

        lax.fori_loop(0, my_ct // 2, pair, 0)

        @pl.when(my_ct % 2 == 1)
        def _():
            c0 = (start + my_ct - 1) * CH
            add_chunk(c0, ia, ma).wait()

        plsc.subcore_barrier()

        @pl.when(cid == 0)
        def _():
            pltpu.sync_copy(acc_spmem.at[pl.ds(sid * rpt, rpt)],
                            agg0_hbm.at[pl.ds(sid * rpt, rpt)])

        @pl.when(cid == 1)
        def _():
            pltpu.sync_copy(acc_spmem.at[pl.ds(sid * rpt, rpt)],
                            agg1_hbm.at[pl.ds(sid * rpt, rpt)])

    return scatter_k


def kernel(x, edge_index, edge_attr, W1, b1, W2, b2, Wih, Whh, bih, bhh):
    n_nodes, h = x.shape
    n_edges = edge_attr.shape[0]
    assert h == H

    src = edge_index[0].astype(jnp.int32)
    dst = edge_index[1].astype(jnp.int32)

    bn = 2000
    be = 2000
    assert n_nodes % bn == 0 and n_edges % be == 0

    f32 = jnp.float32

    s1t, s2t = pl.pallas_call(
        _node_pre_body,
        grid=(n_nodes // bn,),
        in_specs=[pl.BlockSpec((bn, H), lambda i: (i, 0)),
                  pl.BlockSpec((H, H3), lambda i: (0, 0)),
                  pl.BlockSpec((1, H), lambda i: (0, 0))],
        out_specs=[pl.BlockSpec((bn, H), lambda i: (i, 0)),
                   pl.BlockSpec((bn, H), lambda i: (i, 0))],
        out_shape=[jax.ShapeDtypeStruct((n_nodes, H), f32),
                   jax.ShapeDtypeStruct((n_nodes, H), f32)],
    )(x, W1, b1.reshape(1, H))

    nshard = 2
    n_pad = ((n_nodes + NS * 8 - 1) // (NS * 8)) * (NS * 8)
    zeros = jnp.zeros((n_pad // NS, H), f32)
    n_sh = n_edges // nshard
    assert n_sh % CH == 0 and n_sh % be == 0
    sh_blocks = n_sh // be

    aggs = []
    for s in range(nshard):
        off = s * (n_sh // CH)
        g1, g2 = _make_gather(n_nodes, n_sh, off)(s1t, s2t, src, dst)
        msg = pl.pallas_call(
            _edge_mlp_body,
            grid=(sh_blocks,),
            in_specs=[pl.BlockSpec((be, H), lambda i: (i, 0)),
                      pl.BlockSpec((be, H), lambda i: (i, 0)),
                      pl.BlockSpec((be, H),
                                   lambda i, o=s * sh_blocks: (i + o, 0)),
                      pl.BlockSpec((H, H), lambda i: (0, 0)),
                      pl.BlockSpec((H, H), lambda i: (0, 0)),
                      pl.BlockSpec((1, H), lambda i: (0, 0))],
            out_specs=pl.BlockSpec((be, H), lambda i: (i, 0)),
            out_shape=jax.ShapeDtypeStruct((n_sh, H), f32),
        )(g1, g2, edge_attr, W1[:, H2:H3], W2, b2.reshape(1, H))
        a0, a1 = _make_scatter(n_pad, n_sh, off)(msg, dst, zeros)
        aggs += [a0, a1]

    out = pl.pallas_call(
        _make_gru_body(len(aggs)),
        grid=(n_nodes // bn,),
        in_specs=([pl.BlockSpec((bn, H), lambda i: (i, 0))] * (len(aggs) + 1)
                  + [pl.BlockSpec((H3, H), lambda i: (0, 0)),
                     pl.BlockSpec((H3, H), lambda i: (0, 0)),
                     pl.BlockSpec((1, H3), lambda i: (0, 0)),
                     pl.BlockSpec((1, H3), lambda i: (0, 0))]),
        out_specs=pl.BlockSpec((bn, H), lambda i: (i, 0)),
        out_shape=jax.ShapeDtypeStruct((n_nodes, H), f32),
    )(*aggs, x, Wih, Whh, bih.reshape(1, H3), bhh.reshape(1, H3))

    return out


# speedup vs baseline: 1.2061x; 1.0837x over previous
import functools

import jax
import jax.numpy as jnp
from jax import lax
from jax.experimental import pallas as pl
from jax.experimental.pallas import tpu as pltpu
from jax.experimental.pallas import tpu_sc as plsc

H = 128
H2 = 256
H3 = 384

NC = 2
NS = 16
NW = NC * NS

CH = 128

_DN = (((1,), (1,)), ((), ()))


def _node_pre_body(x_ref, w1_ref, b1_ref, s1_ref, s2_ref):
    xb = x_ref[...]
    w1 = w1_ref[...]
    s1_ref[...] = lax.dot_general(xb, w1[:, 0:H], _DN,
                                  preferred_element_type=jnp.float32) + b1_ref[...]
    s2_ref[...] = lax.dot_general(xb, w1[:, H:H2], _DN,
                                  preferred_element_type=jnp.float32)


def _edge_mlp_body(g_ref, ea_ref, w1c_ref, w2_ref, b2_ref, out_ref):
    t = g_ref[...] + lax.dot_general(
        ea_ref[...], w1c_ref[...], _DN, preferred_element_type=jnp.float32)
    h = jnp.maximum(t, 0.0)
    out_ref[...] = lax.dot_general(
        h, w2_ref[...], _DN, preferred_element_type=jnp.float32) + b2_ref[...]


def _make_gru_body(n_agg):
    def _gru_body(*refs):
        agg_refs = refs[:n_agg]
        x_ref, wih_ref, whh_ref, bih_ref, bhh_ref, out_ref = refs[n_agg:]
        agg = functools.reduce(lambda a, r: a + r[...], agg_refs[1:],
                               agg_refs[0][...])
        xb = x_ref[...]
        gi = lax.dot_general(agg, wih_ref[...], _DN,
                             preferred_element_type=jnp.float32) + bih_ref[...]
        gh = lax.dot_general(xb, whh_ref[...], _DN,
                             preferred_element_type=jnp.float32) + bhh_ref[...]
        r = jax.nn.sigmoid(gi[:, 0:H] + gh[:, 0:H])
        z = jax.nn.sigmoid(gi[:, H:H2] + gh[:, H:H2])
        n = jnp.tanh(gi[:, H2:H3] + r * gh[:, H2:H3])
        out_ref[...] = (1.0 - z) * n + z * xb

    return _gru_body


def _chunk_deal(wid, nchunks):
    base_ct = nchunks // NW
    rem = nchunks % NW
    extra = (wid < rem).astype(jnp.int32)
    start = base_ct * wid + jnp.minimum(wid, rem)
    return start, base_ct + extra


def _make_gather(n_nodes, n_edges, chunk_off=0):
    nchunks = n_edges // CH
    assert n_edges % CH == 0
    mesh = plsc.VectorSubcoreMesh(core_axis_name="c", subcore_axis_name="s")

    @functools.partial(
        pl.kernel, mesh=mesh,
        out_type=jax.ShapeDtypeStruct((n_edges, H), jnp.float32),
        scratch_types=[pltpu.VMEM((CH,), jnp.int32),
                       pltpu.VMEM((CH,), jnp.int32),
                       pltpu.VMEM((CH,), jnp.int32),
                       pltpu.VMEM((CH,), jnp.int32),
                       pltpu.VMEM((CH,), jnp.int32),
                       pltpu.VMEM((CH,), jnp.int32),
                       pltpu.VMEM((CH, H), jnp.float32),
                       pltpu.VMEM((CH, H), jnp.float32),
                       pltpu.VMEM((CH, H), jnp.float32),
                       pltpu.VMEM((CH, H), jnp.float32),
                       pltpu.VMEM_SHARED((NS * 2 * CH, H), jnp.float32),
                       pltpu.SemaphoreType.DMA,
                       pltpu.SemaphoreType.DMA,
                       pltpu.SemaphoreType.DMA],
    )
    def gather_k(s1_hbm, s2_hbm, src_hbm, dst_hbm, o_hbm,
                 ident_a, ident_b, ia1, ia2, ib1, ib2, ba1, ba2, bb1, bb2,
                 stage, gsem, asem, wsem):
        cid = lax.axis_index("c")
        sid = lax.axis_index("s")
        wid = sid * NC + cid
        start, my_ct = _chunk_deal(wid, nchunks)

        row_a = sid * 2 * CH
        for j in range(CH // 16):
            ids = lax.iota(jnp.int32, 16) + (16 * j + row_a)
            ident_a[pl.ds(16 * j, 16)] = ids
            ident_b[pl.ds(16 * j, 16)] = ids + CH

        def do_chunk(base, idx1, idx2, buf1, buf2):
            gbase = base + chunk_off * CH
            pltpu.sync_copy(src_hbm.at[pl.ds(gbase, CH)], idx1)
            pltpu.sync_copy(dst_hbm.at[pl.ds(gbase, CH)], idx2)
            g1 = pltpu.async_copy(s1_hbm.at[idx1], buf1, gsem)
            g2 = pltpu.async_copy(s2_hbm.at[idx2], buf2, gsem)
            return g1, g2

        def sum_chunk(buf1, buf2, set_off, idref):
            pltpu.sync_copy(buf1, stage.at[pl.ds(row_a + set_off, CH)])
            return pltpu.async_copy(buf2, stage.at[idref], asem, add=True)

        def put_chunk(base, set_off):
            return pltpu.async_copy(stage.at[pl.ds(row_a + set_off, CH)],
                                    o_hbm.at[pl.ds(base, CH)], wsem)

        def pair(p, carry):
            c0 = (start + 2 * p) * CH
            c1 = c0 + CH
            ga1, ga2 = do_chunk(c0, ia1, ia2, ba1, ba2)
            gb1, gb2 = do_chunk(c1, ib1, ib2, bb1, bb2)
            ga1.wait()
            ga2.wait()
            aa = sum_chunk(ba1, ba2, 0, ident_a)
            gb1.wait()
            gb2.wait()
            ab = sum_chunk(bb1, bb2, CH, ident_b)
            aa.wait()
            wa = put_chunk(c0, 0)
            ab.wait()
            wb = put_chunk(c1, CH)
            wa.wait()
            wb.wait()
            return carry

        lax.fori_loop(0, my_ct // 2, pair, 0)

        @pl.when(my_ct % 2 == 1)
        def _():
            c0 = (start + my_ct - 1) * CH
            g1, g2 = do_chunk(c0, ia1, ia2, ba1, ba2)
            g1.wait()
            g2.wait()
            sum_chunk(ba1, ba2, 0, ident_a).wait()
            put_chunk(c0, 0).wait()

    return gather_k


def _make_scatter(n_pad, n_edges, chunk_off=0):
    nchunks = n_edges // CH
    rpt = n_pad // NS
    assert n_edges % CH == 0 and n_pad % NS == 0 and rpt % 8 == 0
    mesh = plsc.VectorSubcoreMesh(core_axis_name="c", subcore_axis_name="s")

    @functools.partial(
        pl.kernel, mesh=mesh,
        out_type=[jax.ShapeDtypeStruct((n_pad, H), jnp.float32),
                  jax.ShapeDtypeStruct((n_pad, H), jnp.float32)],
        scratch_types=[pltpu.VMEM((CH,), jnp.int32),
                       pltpu.VMEM((CH,), jnp.int32),
                       pltpu.VMEM((CH, H), jnp.float32),
                       pltpu.VMEM((CH, H), jnp.float32),
                       pltpu.VMEM_SHARED((n_pad, H), jnp.float32),
                       pltpu.SemaphoreType.DMA],
    )
    def scatter_k(msg_hbm, dst_hbm, zeros_hbm, agg0_hbm, agg1_hbm,
                  ia, ib, ma, mb, acc_spmem, ssem):
        cid = lax.axis_index("c")
        sid = lax.axis_index("s")
        wid = sid * NC + cid
        start, my_ct = _chunk_deal(wid, nchunks)

        pltpu.sync_copy(zeros_hbm, acc_spmem.at[pl.ds(sid * rpt, rpt)])
        plsc.subcore_barrier()

        def add_chunk(base, idx, mbuf):
            pltpu.sync_copy(dst_hbm.at[pl.ds(base + chunk_off * CH, CH)], idx)
            pltpu.sync_copy(msg_hbm.at[pl.ds(base, CH)], mbuf)
            return pltpu.async_copy(mbuf, acc_spmem.at[idx], ssem, add=True)

        def pair(p, carry):
            c0 = (start + 2 * p) * CH
            aa = add_chunk(c0, ia, ma)
            ab = add_chunk(c0 + CH, ib, mb)
            aa.wait()
            ab.wait()
            return carry

        lax.fori_loop(0, my_ct // 2, pair, 0)

        @pl.when(my_ct % 2 == 1)
        def _():
            c0 = (start + my_ct - 1) * CH
            add_chunk(c0, ia, ma).wait()

        plsc.subcore_barrier()

        @pl.when(cid == 0)
        def _():
            pltpu.sync_copy(acc_spmem.at[pl.ds(sid * rpt, rpt)],
                            agg0_hbm.at[pl.ds(sid * rpt, rpt)])

        @pl.when(cid == 1)
        def _():
            pltpu.sync_copy(acc_spmem.at[pl.ds(sid * rpt, rpt)],
                            agg1_hbm.at[pl.ds(sid * rpt, rpt)])

    return scatter_k


def kernel(x, edge_index, edge_attr, W1, b1, W2, b2, Wih, Whh, bih, bhh):
    n_nodes, h = x.shape
    n_edges = edge_attr.shape[0]
    assert h == H

    src = edge_index[0].astype(jnp.int32)
    dst = edge_index[1].astype(jnp.int32)

    bn = 2000
    be = 2000
    assert n_nodes % bn == 0 and n_edges % be == 0

    f32 = jnp.float32

    s1t, s2t = pl.pallas_call(
        _node_pre_body,
        grid=(n_nodes // bn,),
        in_specs=[pl.BlockSpec((bn, H), lambda i: (i, 0)),
                  pl.BlockSpec((H, H3), lambda i: (0, 0)),
                  pl.BlockSpec((1, H), lambda i: (0, 0))],
        out_specs=[pl.BlockSpec((bn, H), lambda i: (i, 0)),
                   pl.BlockSpec((bn, H), lambda i: (i, 0))],
        out_shape=[jax.ShapeDtypeStruct((n_nodes, H), f32),
                   jax.ShapeDtypeStruct((n_nodes, H), f32)],
    )(x, W1, b1.reshape(1, H))

    nshard = 2
    n_pad = ((n_nodes + NS * 8 - 1) // (NS * 8)) * (NS * 8)
    zeros = jnp.zeros((n_pad // NS, H), f32)
    n_sh = n_edges // nshard
    assert n_sh % CH == 0 and n_sh % be == 0
    sh_blocks = n_sh // be

    aggs = []
    for s in range(nshard):
        off = s * (n_sh // CH)
        g = _make_gather(n_nodes, n_sh, off)(s1t, s2t, src, dst)
        msg = pl.pallas_call(
            _edge_mlp_body,
            grid=(sh_blocks,),
            in_specs=[pl.BlockSpec((be, H), lambda i: (i, 0)),
                      pl.BlockSpec((be, H),
                                   lambda i, o=s * sh_blocks: (i + o, 0)),
                      pl.BlockSpec((H, H), lambda i: (0, 0)),
                      pl.BlockSpec((H, H), lambda i: (0, 0)),
                      pl.BlockSpec((1, H), lambda i: (0, 0))],
            out_specs=pl.BlockSpec((be, H), lambda i: (i, 0)),
            out_shape=jax.ShapeDtypeStruct((n_sh, H), f32),
        )(g, edge_attr, W1[:, H2:H3], W2, b2.reshape(1, H))
        a0, a1 = _make_scatter(n_pad, n_sh, off)(msg, dst, zeros)
        aggs += [a0, a1]

    out = pl.pallas_call(
        _make_gru_body(len(aggs)),
        grid=(n_nodes // bn,),
        in_specs=([pl.BlockSpec((bn, H), lambda i: (i, 0))] * (len(aggs) + 1)
                  + [pl.BlockSpec((H3, H), lambda i: (0, 0)),
                     pl.BlockSpec((H3, H), lambda i: (0, 0)),
                     pl.BlockSpec((1, H3), lambda i: (0, 0)),
                     pl.BlockSpec((1, H3), lambda i: (0, 0))]),
        out_specs=pl.BlockSpec((bn, H), lambda i: (i, 0)),
        out_shape=jax.ShapeDtypeStruct((n_nodes, H), f32),
    )(*aggs, x, Wih, Whh, bih.reshape(1, H3), bhh.reshape(1, H3))

    return out


# speedup vs baseline: 1.2791x; 1.0606x over previous
import functools

import jax
import jax.numpy as jnp
from jax import lax
from jax.experimental import pallas as pl
from jax.experimental.pallas import tpu as pltpu
from jax.experimental.pallas import tpu_sc as plsc

H = 128
H2 = 256
H3 = 384

NC = 2
NS = 16
NW = NC * NS

CH = 128

_DN = (((1,), (1,)), ((), ()))


def _node_pre_body(x_ref, w1_ref, b1_ref, s1_ref, s2_ref):
    xb = x_ref[...]
    w1 = w1_ref[...]
    s1_ref[...] = lax.dot_general(xb, w1[:, 0:H], _DN,
                                  preferred_element_type=jnp.float32) + b1_ref[...]
    s2_ref[...] = lax.dot_general(xb, w1[:, H:H2], _DN,
                                  preferred_element_type=jnp.float32)


def _edge_mlp_body(g_ref, ea_ref, w1c_ref, w2_ref, b2_ref, out_ref):
    t = g_ref[...] + lax.dot_general(
        ea_ref[...], w1c_ref[...], _DN, preferred_element_type=jnp.float32)
    h = jnp.maximum(t, 0.0)
    out_ref[...] = lax.dot_general(
        h, w2_ref[...], _DN, preferred_element_type=jnp.float32) + b2_ref[...]


def _make_gru_body(n_agg):
    def _gru_body(*refs):
        agg_refs = refs[:n_agg]
        x_ref, wih_ref, whh_ref, bih_ref, bhh_ref, out_ref = refs[n_agg:]
        agg = functools.reduce(lambda a, r: a + r[...], agg_refs[1:],
                               agg_refs[0][...])
        xb = x_ref[...]
        gi = lax.dot_general(agg, wih_ref[...], _DN,
                             preferred_element_type=jnp.float32) + bih_ref[...]
        gh = lax.dot_general(xb, whh_ref[...], _DN,
                             preferred_element_type=jnp.float32) + bhh_ref[...]
        r = jax.nn.sigmoid(gi[:, 0:H] + gh[:, 0:H])
        z = jax.nn.sigmoid(gi[:, H:H2] + gh[:, H:H2])
        n = jnp.tanh(gi[:, H2:H3] + r * gh[:, H2:H3])
        out_ref[...] = (1.0 - z) * n + z * xb

    return _gru_body


def _chunk_deal(wid, nchunks):
    base_ct = nchunks // NW
    rem = nchunks % NW
    extra = (wid < rem).astype(jnp.int32)
    start = base_ct * wid + jnp.minimum(wid, rem)
    return start, base_ct + extra


def _make_gather(n_nodes, n_edges, chunk_off=0):
    nchunks = n_edges // CH
    assert n_edges % CH == 0
    mesh = plsc.VectorSubcoreMesh(core_axis_name="c", subcore_axis_name="s")

    @functools.partial(
        pl.kernel, mesh=mesh,
        out_type=jax.ShapeDtypeStruct((n_edges, H), jnp.float32),
        scratch_types=[pltpu.VMEM((CH,), jnp.int32),
                       pltpu.VMEM((CH,), jnp.int32),
                       pltpu.VMEM((CH,), jnp.int32),
                       pltpu.VMEM((CH,), jnp.int32),
                       pltpu.VMEM((CH,), jnp.int32),
                       pltpu.VMEM((CH,), jnp.int32),
                       pltpu.VMEM((CH, H), jnp.float32),
                       pltpu.VMEM((CH, H), jnp.float32),
                       pltpu.VMEM((CH, H), jnp.float32),
                       pltpu.VMEM((CH, H), jnp.float32),
                       pltpu.VMEM_SHARED((NS * 2 * CH, H), jnp.float32),
                       pltpu.SemaphoreType.DMA,
                       pltpu.SemaphoreType.DMA,
                       pltpu.SemaphoreType.DMA],
    )
    def gather_k(s1_hbm, s2_hbm, src_hbm, dst_hbm, o_hbm,
                 ident_a, ident_b, ia1, ia2, ib1, ib2, ba1, ba2, bb1, bb2,
                 stage, gsem, asem, wsem):
        cid = lax.axis_index("c")
        sid = lax.axis_index("s")
        wid = sid * NC + cid
        start, my_ct = _chunk_deal(wid, nchunks)

        row_a = sid * 2 * CH
        for j in range(CH // 16):
            ids = lax.iota(jnp.int32, 16) + (16 * j + row_a)
            ident_a[pl.ds(16 * j, 16)] = ids
            ident_b[pl.ds(16 * j, 16)] = ids + CH

        def do_chunk(base, idx1, idx2, buf1, buf2):
            gbase = base + chunk_off * CH
            pltpu.sync_copy(src_hbm.at[pl.ds(gbase, CH)], idx1)
            pltpu.sync_copy(dst_hbm.at[pl.ds(gbase, CH)], idx2)
            g1 = pltpu.async_copy(s1_hbm.at[idx1], buf1, gsem)
            g2 = pltpu.async_copy(s2_hbm.at[idx2], buf2, gsem)
            return g1, g2

        def sum_chunk(buf1, buf2, set_off, idref):
            pltpu.sync_copy(buf1, stage.at[pl.ds(row_a + set_off, CH)])
            return pltpu.async_copy(buf2, stage.at[idref], asem, add=True)

        def put_chunk(base, set_off):
            return pltpu.async_copy(stage.at[pl.ds(row_a + set_off, CH)],
                                    o_hbm.at[pl.ds(base, CH)], wsem)

        def pair(p, carry):
            c0 = (start + 2 * p) * CH
            c1 = c0 + CH
            ga1, ga2 = do_chunk(c0, ia1, ia2, ba1, ba2)
            gb1, gb2 = do_chunk(c1, ib1, ib2, bb1, bb2)
            ga1.wait()
            ga2.wait()
            aa = sum_chunk(ba1, ba2, 0, ident_a)
            gb1.wait()
            gb2.wait()
            ab = sum_chunk(bb1, bb2, CH, ident_b)
            aa.wait()
            wa = put_chunk(c0, 0)
            ab.wait()
            wb = put_chunk(c1, CH)
            wa.wait()
            wb.wait()
            return carry

        lax.fori_loop(0, my_ct // 2, pair, 0)

        @pl.when(my_ct % 2 == 1)
        def _():
            c0 = (start + my_ct - 1) * CH
            g1, g2 = do_chunk(c0, ia1, ia2, ba1, ba2)
            g1.wait()
            g2.wait()
            sum_chunk(ba1, ba2, 0, ident_a).wait()
            put_chunk(c0, 0).wait()

    return gather_k


def _make_scatter(n_pad, n_edges, chunk_off=0):
    nchunks = n_edges // CH
    rpt = n_pad // NS
    assert n_edges % CH == 0 and n_pad % NS == 0 and rpt % 8 == 0
    mesh = plsc.VectorSubcoreMesh(core_axis_name="c", subcore_axis_name="s")

    @functools.partial(
        pl.kernel, mesh=mesh,
        out_type=[jax.ShapeDtypeStruct((n_pad, H), jnp.float32),
                  jax.ShapeDtypeStruct((n_pad, H), jnp.float32)],
        scratch_types=[pltpu.VMEM((CH,), jnp.int32),
                       pltpu.VMEM((CH,), jnp.int32),
                       pltpu.VMEM((CH,), jnp.int32),
                       pltpu.VMEM((CH, H), jnp.float32),
                       pltpu.VMEM((CH, H), jnp.float32),
                       pltpu.VMEM((CH, H), jnp.float32),
                       pltpu.SemaphoreType.DMA,
                       pltpu.SemaphoreType.DMA,
                       pltpu.VMEM_SHARED((n_pad, H), jnp.float32)],
    )
    def scatter_k(msg_hbm, dst_hbm, zeros_hbm, agg0_hbm, agg1_hbm,
                  ia, ib, ic, ma, mb, mc, lsem, ssem, acc_spmem):
        cid = lax.axis_index("c")
        sid = lax.axis_index("s")
        wid = sid * NC + cid
        start, my_ct = _chunk_deal(wid, nchunks)

        pltpu.sync_copy(zeros_hbm, acc_spmem.at[pl.ds(sid * rpt, rpt)])
        plsc.subcore_barrier()

        def load_chunk(base, idx, mbuf):
            j = pltpu.async_copy(
                dst_hbm.at[pl.ds(base + chunk_off * CH, CH)], idx, lsem)
            l = pltpu.async_copy(msg_hbm.at[pl.ds(base, CH)], mbuf, lsem)
            return j, l

        def fire_add(idx, mbuf):
            return pltpu.async_copy(mbuf, acc_spmem.at[idx], ssem, add=True)

        def trip(t, carry):
            c0 = (start + 3 * t) * CH
            ja, la = load_chunk(c0, ia, ma)
            jb, lb = load_chunk(c0 + CH, ib, mb)
            jc, lc = load_chunk(c0 + 2 * CH, ic, mc)
            ja.wait()
            la.wait()
            aa = fire_add(ia, ma)
            jb.wait()
            lb.wait()
            ab = fire_add(ib, mb)
            aa.wait()
            jc.wait()
            lc.wait()
            ac = fire_add(ic, mc)
            ab.wait()
            ac.wait()
            return carry

        lax.fori_loop(0, my_ct // 3, trip, 0)

        tail = my_ct - (my_ct // 3) * 3

        @pl.when(tail >= 1)
        def _():
            c0 = (start + my_ct - tail) * CH
            ja, la = load_chunk(c0, ia, ma)
            ja.wait()
            la.wait()
            fire_add(ia, ma).wait()

        @pl.when(tail == 2)
        def _():
            c0 = (start + my_ct - 1) * CH
            jb, lb = load_chunk(c0, ib, mb)
            jb.wait()
            lb.wait()
            fire_add(ib, mb).wait()

        plsc.subcore_barrier()

        @pl.when(cid == 0)
        def _():
            pltpu.sync_copy(acc_spmem.at[pl.ds(sid * rpt, rpt)],
                            agg0_hbm.at[pl.ds(sid * rpt, rpt)])

        @pl.when(cid == 1)
        def _():
            pltpu.sync_copy(acc_spmem.at[pl.ds(sid * rpt, rpt)],
                            agg1_hbm.at[pl.ds(sid * rpt, rpt)])

    return scatter_k


def kernel(x, edge_index, edge_attr, W1, b1, W2, b2, Wih, Whh, bih, bhh):
    n_nodes, h = x.shape
    n_edges = edge_attr.shape[0]
    assert h == H

    src = edge_index[0].astype(jnp.int32)
    dst = edge_index[1].astype(jnp.int32)

    bn = 2000
    be = 2000
    assert n_nodes % bn == 0 and n_edges % be == 0

    f32 = jnp.float32

    s1t, s2t = pl.pallas_call(
        _node_pre_body,
        grid=(n_nodes // bn,),
        in_specs=[pl.BlockSpec((bn, H), lambda i: (i, 0)),
                  pl.BlockSpec((H, H3), lambda i: (0, 0)),
                  pl.BlockSpec((1, H), lambda i: (0, 0))],
        out_specs=[pl.BlockSpec((bn, H), lambda i: (i, 0)),
                   pl.BlockSpec((bn, H), lambda i: (i, 0))],
        out_shape=[jax.ShapeDtypeStruct((n_nodes, H), f32),
                   jax.ShapeDtypeStruct((n_nodes, H), f32)],
    )(x, W1, b1.reshape(1, H))

    nshard = 2
    n_pad = ((n_nodes + NS * 8 - 1) // (NS * 8)) * (NS * 8)
    zeros = jnp.zeros((n_pad // NS, H), f32)
    n_sh = n_edges // nshard
    assert n_sh % CH == 0 and n_sh % be == 0
    sh_blocks = n_sh // be

    aggs = []
    for s in range(nshard):
        off = s * (n_sh // CH)
        g = _make_gather(n_nodes, n_sh, off)(s1t, s2t, src, dst)
        msg = pl.pallas_call(
            _edge_mlp_body,
            grid=(sh_blocks,),
            in_specs=[pl.BlockSpec((be, H), lambda i: (i, 0)),
                      pl.BlockSpec((be, H),
                                   lambda i, o=s * sh_blocks: (i + o, 0)),
                      pl.BlockSpec((H, H), lambda i: (0, 0)),
                      pl.BlockSpec((H, H), lambda i: (0, 0)),
                      pl.BlockSpec((1, H), lambda i: (0, 0))],
            out_specs=pl.BlockSpec((be, H), lambda i: (i, 0)),
            out_shape=jax.ShapeDtypeStruct((n_sh, H), f32),
        )(g, edge_attr, W1[:, H2:H3], W2, b2.reshape(1, H))
        a0, a1 = _make_scatter(n_pad, n_sh, off)(msg, dst, zeros)
        aggs += [a0, a1]

    out = pl.pallas_call(
        _make_gru_body(len(aggs)),
        grid=(n_nodes // bn,),
        in_specs=([pl.BlockSpec((bn, H), lambda i: (i, 0))] * (len(aggs) + 1)
                  + [pl.BlockSpec((H3, H), lambda i: (0, 0)),
                     pl.BlockSpec((H3, H), lambda i: (0, 0)),
                     pl.BlockSpec((1, H3), lambda i: (0, 0)),
                     pl.BlockSpec((1, H3), lambda i: (0, 0))]),
        out_specs=pl.BlockSpec((bn, H), lambda i: (i, 0)),
        out_shape=jax.ShapeDtypeStruct((n_nodes, H), f32),
    )(*aggs, x, Wih, Whh, bih.reshape(1, H3), bhh.reshape(1, H3))

    return out
